# baseline (speedup 1.0000x reference)
import functools

import jax
import jax.numpy as jnp
from jax import lax
from jax.experimental import pallas as pl
from jax.experimental.pallas import tpu as pltpu
from jax.experimental.pallas import tpu_sc as plsc

N = 16384
D = 64
DP = 128
K = 5
NC = 2
NS = 16
NW = NC * NS
R = N // NW
CH = 64
NCHUNK = R // CH
L = 16
NL = D // L

SC_OUT = N * (1 + K) * L

_mesh = plsc.VectorSubcoreMesh(core_axis_name="c", subcore_axis_name="s")


@functools.partial(
    pl.kernel,
    mesh=_mesh,
    out_type=jax.ShapeDtypeStruct((SC_OUT,), jnp.float32),
    scratch_types=[
        pltpu.VMEM((CH,), jnp.int32),
        pltpu.VMEM((K, CH), jnp.int32),
        pltpu.VMEM((CH,), jnp.int32),
        pltpu.VMEM((K, CH), jnp.int32),
        pltpu.VMEM((CH, DP), jnp.float32),
        pltpu.VMEM((K * CH, DP), jnp.float32),
        pltpu.VMEM((CH // 2, DP), jnp.float32),
        pltpu.VMEM((CH * L,), jnp.float32),
        pltpu.VMEM((K * CH * L,), jnp.float32),
        pltpu.SemaphoreType.DMA,
    ],
)
def _sc_scores(table, lab_idx, neg_idx, hid, out,
               pidx_v, nidx_v, pidx2_v, nidx2_v, lab_v, neg_v, hid_v,
               ps_v, ns_v, sem):
    w = lax.axis_index("s") * NC + lax.axis_index("c")

    def chunk(c, _):
        base = pl.multiple_of(w * R + c * CH, CH)
        hbase = pl.multiple_of(w * (R // 2) + c * (CH // 2), CH // 2)
        pltpu.sync_copy(lab_idx.at[pl.ds(base, CH)], pidx_v)
        for k in range(K):
            pltpu.sync_copy(neg_idx.at[pl.ds(base * K + k * CH, CH)],
                            nidx_v.at[k])
        for q in range(CH // L):
            pidx2_v[pl.ds(q * L, L)] = pidx_v[pl.ds(q * L, L)] >> 1
            for k in range(K):
                nidx2_v[k, pl.ds(q * L, L)] = nidx_v[k, pl.ds(q * L, L)] >> 1
        copies = [pltpu.async_copy(table.at[pidx2_v], lab_v, sem)]
        for k in range(K):
            copies.append(pltpu.async_copy(table.at[nidx2_v.at[k]],
                                           neg_v.at[pl.ds(k * CH, CH)], sem))
        copies.append(pltpu.async_copy(hid.at[pl.ds(hbase, CH // 2)],
                                       hid_v, sem))
        for cp in copies:
            cp.wait()

        def group(q, _):
            poff = (pidx_v[pl.ds(q * L, L)] & 1) * D
            noff = [(nidx_v[k, pl.ds(q * L, L)] & 1) * D for k in range(K)]
            for r in range(L):
                i = q * L + r
                hrow = q * (L // 2) + r // 2
                hcol = (r % 2) * D
                hs = [hid_v[hrow, pl.ds(hcol + t * L, L)] for t in range(NL)]
                off = poff[r]
                acc = hs[0] * lab_v[i, pl.ds(off, L)]
                for t in range(1, NL):
                    acc = acc + hs[t] * lab_v[i, pl.ds(off + t * L, L)]
                ps_v[pl.ds(i * L, L)] = acc
                for k in range(K):
                    j = i * K + k
                    off = noff[k][r]
                    acc = hs[0] * neg_v[j, pl.ds(off, L)]
                    for t in range(1, NL):
                        acc = acc + hs[t] * neg_v[j, pl.ds(off + t * L, L)]
                    ns_v[pl.ds(j * L, L)] = acc
            return 0

        lax.fori_loop(0, CH // L, group, 0)
        pltpu.sync_copy(ps_v, out.at[pl.ds(base * L, CH * L)])
        pltpu.sync_copy(ns_v, out.at[pl.ds((N + base * K) * L, K * CH * L)])
        return 0

    lax.fori_loop(0, NCHUNK, chunk, 0)


TC_ROWS = SC_OUT // 128
TC_BLK = 2048


def _tc_body(s_ref, o_ref):
    b = pl.program_id(0)
    pos = b == 0
    sgn = jnp.where(pos, 1.0, -1.0)
    wgt = jnp.where(pos, 1.0, 1.0 / K)
    x = s_ref[...]
    acc = jnp.float32(0.0)
    for g in range(128 // L):
        score = jnp.sum(x[:, g * L:(g + 1) * L], axis=1, keepdims=True)
        y = sgn * score
        ls = jnp.minimum(y, 0.0) - jnp.log1p(jnp.exp(-jnp.abs(y)))
        acc = acc + jnp.sum(ls)

    @pl.when(b == 0)
    def _():
        o_ref[0, 0] = 0.0

    o_ref[0, 0] = o_ref[0, 0] - wgt * acc


_tc_loss = pl.pallas_call(
    _tc_body,
    grid=(TC_ROWS // TC_BLK,),
    in_specs=[pl.BlockSpec((TC_BLK, 128), lambda i: (i, 0))],
    out_specs=pl.BlockSpec(memory_space=pltpu.SMEM, block_shape=(1, 1),
                           index_map=lambda i: (0, 0)),
    out_shape=jax.ShapeDtypeStruct((1, 1), jnp.float32),
    compiler_params=pltpu.CompilerParams(
        dimension_semantics=("arbitrary",)),
)


def kernel(hidden_state, label_idxes, neg_idxes, out_word_emb):
    lab = label_idxes.astype(jnp.int32)
    neg = neg_idxes.astype(jnp.int32)
    table2 = out_word_emb.reshape(-1, DP)
    hid2 = hidden_state.reshape(N // 2, DP)
    partials = _sc_scores(table2, lab, neg, hid2)
    loss = _tc_loss(partials.reshape(TC_ROWS, 128))
    return loss.reshape(())


# speedup vs baseline: 1.8471x; 1.8471x over previous
import functools

import jax
import jax.numpy as jnp
from jax import lax
from jax.experimental import pallas as pl
from jax.experimental.pallas import tpu as pltpu
from jax.experimental.pallas import tpu_sc as plsc

N = 16384
D = 64
DP = 128
K = 5
V = 1000000
NC = 2
NS = 16
NW = NC * NS
R = N // NW
CH = 64
NCHUNK = R // CH
L = 16
NL = D // L

SC_OUT = N * (1 + K) * L

H = 1 << 19
TW = 4096
T_GRID = H // TW
_HI_MAX = V // TW - 1


def _repack_body(lo_ref, hi_ref, o_ref):
    xlo = jnp.transpose(lo_ref[...], (1, 0))
    xhi = jnp.transpose(hi_ref[...], (1, 0))
    o_ref[...] = jnp.concatenate([xlo, xhi], axis=1)


_repack = pl.pallas_call(
    _repack_body,
    grid=(T_GRID,),
    in_specs=[pl.BlockSpec((D, TW), lambda i: (0, i)),
              pl.BlockSpec((D, TW),
                           lambda i: (0, jnp.minimum(i + T_GRID, _HI_MAX)))],
    out_specs=pl.BlockSpec((TW, DP), lambda i: (i, 0)),
    out_shape=jax.ShapeDtypeStruct((H, DP), jnp.float32),
    compiler_params=pltpu.CompilerParams(
        dimension_semantics=("arbitrary",)),
)

_mesh = plsc.VectorSubcoreMesh(core_axis_name="c", subcore_axis_name="s")


@functools.partial(
    pl.kernel,
    mesh=_mesh,
    out_type=jax.ShapeDtypeStruct((SC_OUT,), jnp.float32),
    scratch_types=[
        pltpu.VMEM((CH,), jnp.int32),
        pltpu.VMEM((K, CH), jnp.int32),
        pltpu.VMEM((CH,), jnp.int32),
        pltpu.VMEM((K, CH), jnp.int32),
        pltpu.VMEM((CH, DP), jnp.float32),
        pltpu.VMEM((K * CH, DP), jnp.float32),
        pltpu.VMEM((CH // 2, DP), jnp.float32),
        pltpu.VMEM((CH * L,), jnp.float32),
        pltpu.VMEM((K * CH * L,), jnp.float32),
        pltpu.SemaphoreType.DMA,
    ],
)
def _sc_scores(table, lab_idx, neg_idx, lab_gidx, neg_gidx, hid, out,
               pidx_v, nidx_v, pidx2_v, nidx2_v, lab_v, neg_v, hid_v,
               ps_v, ns_v, sem):
    w = lax.axis_index("s") * NC + lax.axis_index("c")

    def chunk(c, _):
        base = pl.multiple_of(w * R + c * CH, CH)
        hbase = pl.multiple_of(w * (R // 2) + c * (CH // 2), CH // 2)
        pltpu.sync_copy(lab_idx.at[pl.ds(base, CH)], pidx_v)
        pltpu.sync_copy(lab_gidx.at[pl.ds(base, CH)], pidx2_v)
        for k in range(K):
            pltpu.sync_copy(neg_idx.at[pl.ds(base * K + k * CH, CH)],
                            nidx_v.at[k])
            pltpu.sync_copy(neg_gidx.at[pl.ds(base * K + k * CH, CH)],
                            nidx2_v.at[k])
        copies = [pltpu.async_copy(table.at[pidx2_v], lab_v, sem)]
        for k in range(K):
            copies.append(pltpu.async_copy(table.at[nidx2_v.at[k]],
                                           neg_v.at[pl.ds(k * CH, CH)], sem))
        copies.append(pltpu.async_copy(hid.at[pl.ds(hbase, CH // 2)],
                                       hid_v, sem))
        for cp in copies:
            cp.wait()

        def group(q, _):
            poff = (pidx_v[pl.ds(q * L, L)] >> 19) * D
            noff = [(nidx_v[k, pl.ds(q * L, L)] >> 19) * D for k in range(K)]
            for r in range(L):
                i = q * L + r
                hrow = q * (L // 2) + r // 2
                hcol = (r % 2) * D
                hs = [hid_v[hrow, pl.ds(hcol + t * L, L)] for t in range(NL)]
                off = poff[r]
                acc = hs[0] * lab_v[i, pl.ds(off, L)]
                for t in range(1, NL):
                    acc = acc + hs[t] * lab_v[i, pl.ds(off + t * L, L)]
                ps_v[pl.ds(i * L, L)] = acc
                for k in range(K):
                    j = i * K + k
                    off = noff[k][r]
                    acc = hs[0] * neg_v[j, pl.ds(off, L)]
                    for t in range(1, NL):
                        acc = acc + hs[t] * neg_v[j, pl.ds(off + t * L, L)]
                    ns_v[pl.ds(j * L, L)] = acc
            return 0

        lax.fori_loop(0, CH // L, group, 0)
        pltpu.sync_copy(ps_v, out.at[pl.ds(base * L, CH * L)])
        pltpu.sync_copy(ns_v, out.at[pl.ds((N + base * K) * L, K * CH * L)])
        return 0

    lax.fori_loop(0, NCHUNK, chunk, 0)


TC_ROWS = SC_OUT // 128
TC_BLK = 2048


def _tc_body(s_ref, o_ref):
    b = pl.program_id(0)
    pos = b == 0
    sgn = jnp.where(pos, 1.0, -1.0)
    wgt = jnp.where(pos, 1.0, 1.0 / K)
    x = s_ref[...]
    lane = lax.broadcasted_iota(jnp.int32, (128, 8), 0)
    grp = lax.broadcasted_iota(jnp.int32, (128, 8), 1)
    m = (lane // L == grp).astype(jnp.float32)
    score = lax.dot_general(x, m, (((1,), (0,)), ((), ())),
                            preferred_element_type=jnp.float32)
    y = sgn * score
    ls = jnp.minimum(y, 0.0) - jnp.log1p(jnp.exp(-jnp.abs(y)))

    @pl.when(b == 0)
    def _():
        o_ref[0, 0] = 0.0

    o_ref[0, 0] = o_ref[0, 0] - wgt * jnp.sum(ls)


_tc_loss = pl.pallas_call(
    _tc_body,
    grid=(TC_ROWS // TC_BLK,),
    in_specs=[pl.BlockSpec((TC_BLK, 128), lambda i: (i, 0))],
    out_specs=pl.BlockSpec(memory_space=pltpu.SMEM, block_shape=(1, 1),
                           index_map=lambda i: (0, 0)),
    out_shape=jax.ShapeDtypeStruct((1, 1), jnp.float32),
    compiler_params=pltpu.CompilerParams(
        dimension_semantics=("arbitrary",)),
)


def kernel(hidden_state, label_idxes, neg_idxes, out_word_emb):
    lab = label_idxes.astype(jnp.int32)
    neg = neg_idxes.astype(jnp.int32)
    table2 = _repack(out_word_emb.T, out_word_emb.T)
    hid2 = hidden_state.reshape(N // 2, DP)
    partials = _sc_scores(table2, lab, neg, lab & (H - 1), neg & (H - 1),
                          hid2)
    loss = _tc_loss(partials.reshape(TC_ROWS, 128))
    return loss.reshape(())


# speedup vs baseline: 1.9974x; 1.0814x over previous
import functools

import jax
import jax.numpy as jnp
from jax import lax
from jax.experimental import pallas as pl
from jax.experimental.pallas import tpu as pltpu
from jax.experimental.pallas import tpu_sc as plsc

N = 16384
D = 64
DP = 128
K = 5
V = 1000000
NC = 2
NS = 16
NW = NC * NS
R = N // NW
CH = 64
NCHUNK = R // CH
L = 16
NL = D // L

SC_OUT = N * (1 + K) * L

H = 1 << 19
TW = 8192
T_GRID = H // TW
_HI_MAX = V // TW - 1


def _eye64():
    a = lax.broadcasted_iota(jnp.int32, (D, D), 0)
    b = lax.broadcasted_iota(jnp.int32, (D, D), 1)
    return (a == b).astype(jnp.float32)


def _mxu_t(x, eye):
    return lax.dot_general(x, eye, (((0,), (0,)), ((), ())),
                           preferred_element_type=jnp.float32)


def _repack_body(lo_ref, hi_ref, o_ref):
    eye = _eye64()
    xlo = _mxu_t(lo_ref[...], eye)
    xhi = _mxu_t(hi_ref[...], eye)
    o_ref[...] = jnp.concatenate([xlo, xhi], axis=1)


_repack = pl.pallas_call(
    _repack_body,
    grid=(T_GRID,),
    in_specs=[pl.BlockSpec((D, TW), lambda i: (0, i)),
              pl.BlockSpec((D, TW),
                           lambda i: (0, jnp.minimum(i + T_GRID, _HI_MAX)))],
    out_specs=pl.BlockSpec((TW, DP), lambda i: (i, 0)),
    out_shape=jax.ShapeDtypeStruct((H, DP), jnp.float32),
    compiler_params=pltpu.CompilerParams(
        dimension_semantics=("arbitrary",)),
)

_mesh = plsc.VectorSubcoreMesh(core_axis_name="c", subcore_axis_name="s")


@functools.partial(
    pl.kernel,
    mesh=_mesh,
    out_type=jax.ShapeDtypeStruct((SC_OUT,), jnp.float32),
    scratch_types=[
        pltpu.VMEM((CH,), jnp.int32),
        pltpu.VMEM((K, CH), jnp.int32),
        pltpu.VMEM((CH,), jnp.int32),
        pltpu.VMEM((K, CH), jnp.int32),
        pltpu.VMEM((CH, DP), jnp.float32),
        pltpu.VMEM((K * CH, DP), jnp.float32),
        pltpu.VMEM((CH // 2, DP), jnp.float32),
        pltpu.VMEM((CH * L,), jnp.float32),
        pltpu.VMEM((K * CH * L,), jnp.float32),
        pltpu.SemaphoreType.DMA,
    ],
)
def _sc_scores(table, lab_idx, neg_idx, lab_gidx, neg_gidx, hid, out,
               pidx_v, nidx_v, pidx2_v, nidx2_v, lab_v, neg_v, hid_v,
               ps_v, ns_v, sem):
    w = lax.axis_index("s") * NC + lax.axis_index("c")

    def chunk(c, _):
        base = pl.multiple_of(w * R + c * CH, CH)
        hbase = pl.multiple_of(w * (R // 2) + c * (CH // 2), CH // 2)
        pltpu.sync_copy(lab_idx.at[pl.ds(base, CH)], pidx_v)
        pltpu.sync_copy(lab_gidx.at[pl.ds(base, CH)], pidx2_v)
        for k in range(K):
            pltpu.sync_copy(neg_idx.at[pl.ds(base * K + k * CH, CH)],
                            nidx_v.at[k])
            pltpu.sync_copy(neg_gidx.at[pl.ds(base * K + k * CH, CH)],
                            nidx2_v.at[k])
        copies = [pltpu.async_copy(table.at[pidx2_v], lab_v, sem)]
        for k in range(K):
            copies.append(pltpu.async_copy(table.at[nidx2_v.at[k]],
                                           neg_v.at[pl.ds(k * CH, CH)], sem))
        copies.append(pltpu.async_copy(hid.at[pl.ds(hbase, CH // 2)],
                                       hid_v, sem))
        for cp in copies:
            cp.wait()

        def group(q, _):
            poff = (pidx_v[pl.ds(q * L, L)] >> 19) * D
            noff = [(nidx_v[k, pl.ds(q * L, L)] >> 19) * D for k in range(K)]
            for r in range(L):
                i = q * L + r
                hrow = q * (L // 2) + r // 2
                hcol = (r % 2) * D
                hs = [hid_v[hrow, pl.ds(hcol + t * L, L)] for t in range(NL)]
                off = poff[r]
                acc = hs[0] * lab_v[i, pl.ds(off, L)]
                for t in range(1, NL):
                    acc = acc + hs[t] * lab_v[i, pl.ds(off + t * L, L)]
                ps_v[pl.ds(i * L, L)] = acc
                for k in range(K):
                    j = i * K + k
                    off = noff[k][r]
                    acc = hs[0] * neg_v[j, pl.ds(off, L)]
                    for t in range(1, NL):
                        acc = acc + hs[t] * neg_v[j, pl.ds(off + t * L, L)]
                    ns_v[pl.ds(j * L, L)] = acc
            return 0

        lax.fori_loop(0, CH // L, group, 0)
        pltpu.sync_copy(ps_v, out.at[pl.ds(base * L, CH * L)])
        pltpu.sync_copy(ns_v, out.at[pl.ds((N + base * K) * L, K * CH * L)])
        return 0

    lax.fori_loop(0, NCHUNK, chunk, 0)


TC_ROWS = SC_OUT // 128
TC_BLK = 2048


def _tc_body(s_ref, o_ref):
    b = pl.program_id(0)
    pos = b == 0
    sgn = jnp.where(pos, 1.0, -1.0)
    wgt = jnp.where(pos, 1.0, 1.0 / K)
    x = s_ref[...]
    lane = lax.broadcasted_iota(jnp.int32, (128, 8), 0)
    grp = lax.broadcasted_iota(jnp.int32, (128, 8), 1)
    m = (lane // L == grp).astype(jnp.float32)
    score = lax.dot_general(x, m, (((1,), (0,)), ((), ())),
                            precision=lax.Precision.HIGHEST,
                            preferred_element_type=jnp.float32)
    y = sgn * score
    ls = jnp.minimum(y, 0.0) - jnp.log1p(jnp.exp(-jnp.abs(y)))

    @pl.when(b == 0)
    def _():
        o_ref[0, 0] = 0.0

    o_ref[0, 0] = o_ref[0, 0] - wgt * jnp.sum(ls)


_tc_loss = pl.pallas_call(
    _tc_body,
    grid=(TC_ROWS // TC_BLK,),
    in_specs=[pl.BlockSpec((TC_BLK, 128), lambda i: (i, 0))],
    out_specs=pl.BlockSpec(memory_space=pltpu.SMEM, block_shape=(1, 1),
                           index_map=lambda i: (0, 0)),
    out_shape=jax.ShapeDtypeStruct((1, 1), jnp.float32),
    compiler_params=pltpu.CompilerParams(
        dimension_semantics=("arbitrary",)),
)


def kernel(hidden_state, label_idxes, neg_idxes, out_word_emb):
    lab = label_idxes.astype(jnp.int32)
    neg = neg_idxes.astype(jnp.int32)
    table2 = _repack(out_word_emb.T, out_word_emb.T)
    hid2 = hidden_state.reshape(N // 2, DP)
    partials = _sc_scores(table2, lab, neg, lab & (H - 1), neg & (H - 1),
                          hid2)
    loss = _tc_loss(partials.reshape(TC_ROWS, 128))
    return loss.reshape(())


# speedup vs baseline: 2.0385x; 1.0206x over previous
import functools

import jax
import jax.numpy as jnp
from jax import lax
from jax.experimental import pallas as pl
from jax.experimental.pallas import tpu as pltpu
from jax.experimental.pallas import tpu_sc as plsc

N = 16384
D = 64
DP = 128
K = 5
V = 1000000
NC = 2
NS = 16
NW = NC * NS
R = N // NW
CH = 64
NCHUNK = R // CH
L = 16
NL = DP // L

SC_OUT = N * (1 + K) * 2 * L

H = 1 << 19
TW = 8192
T_GRID = H // TW
_HI_MAX = V // TW - 1


def _eye64():
    a = lax.broadcasted_iota(jnp.int32, (D, D), 0)
    b = lax.broadcasted_iota(jnp.int32, (D, D), 1)
    return (a == b).astype(jnp.float32)


def _mxu_t(x, eye):
    return lax.dot_general(x, eye, (((0,), (0,)), ((), ())),
                           preferred_element_type=jnp.float32)


def _repack_body(lo_ref, hi_ref, o_ref):
    eye = _eye64()
    xlo = _mxu_t(lo_ref[...], eye)
    xhi = _mxu_t(hi_ref[...], eye)
    o_ref[...] = jnp.concatenate([xlo, xhi], axis=1)


_repack = pl.pallas_call(
    _repack_body,
    grid=(T_GRID,),
    in_specs=[pl.BlockSpec((D, TW), lambda i: (0, i)),
              pl.BlockSpec((D, TW),
                           lambda i: (0, jnp.minimum(i + T_GRID, _HI_MAX)))],
    out_specs=pl.BlockSpec((TW, DP), lambda i: (i, 0)),
    out_shape=jax.ShapeDtypeStruct((H, DP), jnp.float32),
    compiler_params=pltpu.CompilerParams(
        dimension_semantics=("arbitrary",)),
)

_mesh = plsc.VectorSubcoreMesh(core_axis_name="c", subcore_axis_name="s")


@functools.partial(
    pl.kernel,
    mesh=_mesh,
    out_type=jax.ShapeDtypeStruct((SC_OUT,), jnp.float32),
    scratch_types=[
        pltpu.VMEM((CH,), jnp.int32),
        pltpu.VMEM((K, CH), jnp.int32),
        pltpu.VMEM((CH, DP), jnp.float32),
        pltpu.VMEM((K * CH, DP), jnp.float32),
        pltpu.VMEM((CH, DP), jnp.float32),
        pltpu.VMEM((CH * 2 * L,), jnp.float32),
        pltpu.VMEM((K * CH * 2 * L,), jnp.float32),
        pltpu.SemaphoreType.DMA,
    ],
)
def _sc_scores(table, lab_gidx, neg_gidx, hid, out,
               pidx_v, nidx_v, lab_v, neg_v, hid_v, ps_v, ns_v, sem):
    w = lax.axis_index("s") * NC + lax.axis_index("c")

    def chunk(c, _):
        base = pl.multiple_of(w * R + c * CH, CH)
        pltpu.sync_copy(lab_gidx.at[pl.ds(base, CH)], pidx_v)
        for k in range(K):
            pltpu.sync_copy(neg_gidx.at[pl.ds(base * K + k * CH, CH)],
                            nidx_v.at[k])
        copies = [pltpu.async_copy(table.at[pidx_v], lab_v, sem)]
        for k in range(K):
            copies.append(pltpu.async_copy(table.at[nidx_v.at[k]],
                                           neg_v.at[pl.ds(k * CH, CH)], sem))
        copies.append(pltpu.async_copy(hid.at[pl.ds(base, CH)], hid_v, sem))
        for cp in copies:
            cp.wait()

        def row(i, _):
            hs = [hid_v[i, pl.ds(t * L, L)] for t in range(NL)]

            def dots(e_ref, j):
                lo = hs[0] * e_ref[j, pl.ds(0, L)]
                hi = hs[4] * e_ref[j, pl.ds(4 * L, L)]
                for t in range(1, NL // 2):
                    lo = lo + hs[t] * e_ref[j, pl.ds(t * L, L)]
                    hi = hi + hs[t + 4] * e_ref[j, pl.ds((t + 4) * L, L)]
                return lo, hi

            lo, hi = dots(lab_v, i)
            ps_v[pl.ds(i * 2 * L, L)] = lo
            ps_v[pl.ds(i * 2 * L + L, L)] = hi
            for k in range(K):
                j = i * K + k
                lo, hi = dots(neg_v, j)
                ns_v[pl.ds(j * 2 * L, L)] = lo
                ns_v[pl.ds(j * 2 * L + L, L)] = hi
            return 0

        lax.fori_loop(0, CH, row, 0)
        pltpu.sync_copy(ps_v, out.at[pl.ds(base * 2 * L, CH * 2 * L)])
        pltpu.sync_copy(
            ns_v, out.at[pl.ds((N + base * K) * 2 * L, K * CH * 2 * L)])
        return 0

    lax.fori_loop(0, NCHUNK, chunk, 0)


TC_ROWS = SC_OUT // 128
TC_BLK = 4096
SPR = 128 // (2 * L)


def _sel_mats():
    lane = lax.broadcasted_iota(jnp.int32, (128, SPR), 0)
    grp = lax.broadcasted_iota(jnp.int32, (128, SPR), 1)
    in_score = (lane // (2 * L)) == grp
    is_hi = (lane // L) % 2 == 1
    m_lo = (in_score & ~is_hi).astype(jnp.float32)
    m_hi = (in_score & is_hi).astype(jnp.float32)
    return m_lo, m_hi


def _tc_body(s_ref, par_ref, o_ref):
    b = pl.program_id(0)
    pos = b == 0
    sgn = jnp.where(pos, 1.0, -1.0)
    wgt = jnp.where(pos, 1.0, 1.0 / K)
    x = s_ref[...]
    m_lo, m_hi = _sel_mats()
    lo = lax.dot_general(x, m_lo, (((1,), (0,)), ((), ())),
                         preferred_element_type=jnp.float32)
    hi = lax.dot_general(x, m_hi, (((1,), (0,)), ((), ())),
                         preferred_element_type=jnp.float32)
    par = par_ref[...].astype(jnp.float32)
    score = lo + par * (hi - lo)
    y = sgn * score
    ls = jnp.minimum(y, 0.0) - jnp.log1p(jnp.exp(-jnp.abs(y)))

    @pl.when(b == 0)
    def _():
        o_ref[0, 0] = 0.0

    o_ref[0, 0] = o_ref[0, 0] - wgt * jnp.sum(ls)


_tc_loss = pl.pallas_call(
    _tc_body,
    grid=(TC_ROWS // TC_BLK,),
    in_specs=[pl.BlockSpec((TC_BLK, 128), lambda i: (i, 0)),
              pl.BlockSpec((TC_BLK, SPR), lambda i: (i, 0))],
    out_specs=pl.BlockSpec(memory_space=pltpu.SMEM, block_shape=(1, 1),
                           index_map=lambda i: (0, 0)),
    out_shape=jax.ShapeDtypeStruct((1, 1), jnp.float32),
    compiler_params=pltpu.CompilerParams(
        dimension_semantics=("arbitrary",)),
)


def kernel(hidden_state, label_idxes, neg_idxes, out_word_emb):
    lab = label_idxes.astype(jnp.int32)
    neg = neg_idxes.astype(jnp.int32)
    table2 = _repack(out_word_emb.T, out_word_emb.T)
    hid3 = jnp.concatenate([hidden_state, hidden_state], axis=1)
    partials = _sc_scores(table2, lab & (H - 1), neg & (H - 1), hid3)
    par = jnp.concatenate([lab >> 19, neg >> 19]).reshape(TC_ROWS, SPR)
    loss = _tc_loss(partials.reshape(TC_ROWS, 128), par)
    return loss.reshape(())


# speedup vs baseline: 2.1089x; 1.0346x over previous
import functools

import jax
import jax.numpy as jnp
from jax import lax
from jax.experimental import pallas as pl
from jax.experimental.pallas import tpu as pltpu
from jax.experimental.pallas import tpu_sc as plsc

N = 16384
D = 64
DP = 128
K = 5
V = 1000000
NC = 2
NS = 16
NW = NC * NS
R = N // NW
CH = 64
NCHUNK = R // CH
L = 16
NL = DP // L

SC_OUT = N * (1 + K) * 2 * L

H = 1 << 19
TW = 16384
T_GRID = H // TW
_HI_MAX = V // TW - 1


def _eye64():
    a = lax.broadcasted_iota(jnp.int32, (D, D), 0)
    b = lax.broadcasted_iota(jnp.int32, (D, D), 1)
    return (a == b).astype(jnp.float32)


def _mxu_t(x, eye):
    return lax.dot_general(x, eye, (((0,), (0,)), ((), ())),
                           preferred_element_type=jnp.float32)


def _repack_body(lo_ref, hi_ref, o_ref):
    eye = _eye64()
    xlo = _mxu_t(lo_ref[...], eye)
    xhi = _mxu_t(hi_ref[...], eye)
    o_ref[...] = jnp.concatenate([xlo, xhi], axis=1)


_repack = pl.pallas_call(
    _repack_body,
    grid=(T_GRID,),
    in_specs=[pl.BlockSpec((D, TW), lambda i: (0, i)),
              pl.BlockSpec((D, TW),
                           lambda i: (0, jnp.minimum(i + T_GRID, _HI_MAX)))],
    out_specs=pl.BlockSpec((TW, DP), lambda i: (i, 0)),
    out_shape=jax.ShapeDtypeStruct((H, DP), jnp.float32),
    compiler_params=pltpu.CompilerParams(
        dimension_semantics=("arbitrary",)),
)

_mesh = plsc.VectorSubcoreMesh(core_axis_name="c", subcore_axis_name="s")


@functools.partial(
    pl.kernel,
    mesh=_mesh,
    out_type=jax.ShapeDtypeStruct((SC_OUT,), jnp.float32),
    scratch_types=[
        pltpu.VMEM((CH,), jnp.int32),
        pltpu.VMEM((K, CH), jnp.int32),
        pltpu.VMEM((CH, DP), jnp.float32),
        pltpu.VMEM((K * CH, DP), jnp.float32),
        pltpu.VMEM((CH, DP), jnp.float32),
        pltpu.VMEM((CH * 2 * L,), jnp.float32),
        pltpu.VMEM((K * CH * 2 * L,), jnp.float32),
        pltpu.SemaphoreType.DMA,
    ],
)
def _sc_scores(table, lab_gidx, neg_gidx, hid, out,
               pidx_v, nidx_v, lab_v, neg_v, hid_v, ps_v, ns_v, sem):
    w = lax.axis_index("s") * NC + lax.axis_index("c")

    def chunk(c, _):
        base = pl.multiple_of(w * R + c * CH, CH)
        pltpu.sync_copy(lab_gidx.at[pl.ds(base, CH)], pidx_v)
        for k in range(K):
            pltpu.sync_copy(neg_gidx.at[pl.ds(base * K + k * CH, CH)],
                            nidx_v.at[k])
        copies = [pltpu.async_copy(table.at[pidx_v], lab_v, sem)]
        for k in range(K):
            copies.append(pltpu.async_copy(table.at[nidx_v.at[k]],
                                           neg_v.at[pl.ds(k * CH, CH)], sem))
        copies.append(pltpu.async_copy(hid.at[pl.ds(base, CH)], hid_v, sem))
        for cp in copies:
            cp.wait()

        def one_row(i):
            hs = [hid_v[i, pl.ds(t * L, L)] for t in range(NL)]

            def dots(e_ref, j):
                p = [hs[t] * e_ref[j, pl.ds(t * L, L)] for t in range(NL)]
                lo = (p[0] + p[1]) + (p[2] + p[3])
                hi = (p[4] + p[5]) + (p[6] + p[7])
                return lo, hi

            lo, hi = dots(lab_v, i)
            ps_v[pl.ds(i * 2 * L, L)] = lo
            ps_v[pl.ds(i * 2 * L + L, L)] = hi
            for k in range(K):
                j = i * K + k
                lo, hi = dots(neg_v, j)
                ns_v[pl.ds(j * 2 * L, L)] = lo
                ns_v[pl.ds(j * 2 * L + L, L)] = hi

        def row2(i2, _):
            one_row(i2 * 2)
            one_row(i2 * 2 + 1)
            return 0

        lax.fori_loop(0, CH // 2, row2, 0)
        pltpu.sync_copy(ps_v, out.at[pl.ds(base * 2 * L, CH * 2 * L)])
        pltpu.sync_copy(
            ns_v, out.at[pl.ds((N + base * K) * 2 * L, K * CH * 2 * L)])
        return 0

    lax.fori_loop(0, NCHUNK, chunk, 0)


TC_ROWS = SC_OUT // 128
TC_BLK = 4096
SPR = 128 // (2 * L)


def _sel_mats():
    lane = lax.broadcasted_iota(jnp.int32, (128, SPR), 0)
    grp = lax.broadcasted_iota(jnp.int32, (128, SPR), 1)
    in_score = (lane // (2 * L)) == grp
    is_hi = (lane // L) % 2 == 1
    m_lo = (in_score & ~is_hi).astype(jnp.float32)
    m_hi = (in_score & is_hi).astype(jnp.float32)
    return m_lo, m_hi


def _tc_body(s_ref, par_ref, o_ref):
    b = pl.program_id(0)
    pos = b == 0
    sgn = jnp.where(pos, 1.0, -1.0)
    wgt = jnp.where(pos, 1.0, 1.0 / K)
    x = s_ref[...]
    m_lo, m_hi = _sel_mats()
    lo = lax.dot_general(x, m_lo, (((1,), (0,)), ((), ())),
                         preferred_element_type=jnp.float32)
    hi = lax.dot_general(x, m_hi, (((1,), (0,)), ((), ())),
                         preferred_element_type=jnp.float32)
    par = par_ref[...].astype(jnp.float32)
    score = lo + par * (hi - lo)
    y = sgn * score
    ls = jnp.minimum(y, 0.0) - jnp.log1p(jnp.exp(-jnp.abs(y)))

    @pl.when(b == 0)
    def _():
        o_ref[0, 0] = 0.0

    o_ref[0, 0] = o_ref[0, 0] - wgt * jnp.sum(ls)


_tc_loss = pl.pallas_call(
    _tc_body,
    grid=(TC_ROWS // TC_BLK,),
    in_specs=[pl.BlockSpec((TC_BLK, 128), lambda i: (i, 0)),
              pl.BlockSpec((TC_BLK, SPR), lambda i: (i, 0))],
    out_specs=pl.BlockSpec(memory_space=pltpu.SMEM, block_shape=(1, 1),
                           index_map=lambda i: (0, 0)),
    out_shape=jax.ShapeDtypeStruct((1, 1), jnp.float32),
    compiler_params=pltpu.CompilerParams(
        dimension_semantics=("arbitrary",)),
)


def kernel(hidden_state, label_idxes, neg_idxes, out_word_emb):
    lab = label_idxes.astype(jnp.int32)
    neg = neg_idxes.astype(jnp.int32)
    table2 = _repack(out_word_emb.T, out_word_emb.T)
    hid3 = jnp.concatenate([hidden_state, hidden_state], axis=1)
    partials = _sc_scores(table2, lab & (H - 1), neg & (H - 1), hid3)
    par = jnp.concatenate([lab >> 19, neg >> 19]).reshape(TC_ROWS, SPR)
    loss = _tc_loss(partials.reshape(TC_ROWS, 128), par)
    return loss.reshape(())


# speedup vs baseline: 2.3203x; 1.1002x over previous
import functools

import jax
import jax.numpy as jnp
from jax import lax
from jax.experimental import pallas as pl
from jax.experimental.pallas import tpu as pltpu
from jax.experimental.pallas import tpu_sc as plsc

N = 16384
D = 64
DP = 128
K = 5
V = 1000000
NC = 2
NS = 16
NW = NC * NS
R = N // NW
CH = 32
NCHUNK = R // CH
IW = (K + 1) * CH
L = 16
NL = DP // L

SC_OUT = N * (1 + K) * 2 * L

H = 1 << 19
TW = 16384
T_GRID = H // TW
_HI_MAX = V // TW - 1


def _eye64():
    a = lax.broadcasted_iota(jnp.int32, (D, D), 0)
    b = lax.broadcasted_iota(jnp.int32, (D, D), 1)
    return (a == b).astype(jnp.float32)


def _mxu_t(x, eye):
    return lax.dot_general(x, eye, (((0,), (0,)), ((), ())),
                           preferred_element_type=jnp.float32)


def _repack_body(lo_ref, hi_ref, o_ref):
    eye = _eye64()
    xlo = _mxu_t(lo_ref[...], eye)
    xhi = _mxu_t(hi_ref[...], eye)
    o_ref[...] = jnp.concatenate([xlo, xhi], axis=1)


_repack = pl.pallas_call(
    _repack_body,
    grid=(T_GRID,),
    in_specs=[pl.BlockSpec((D, TW), lambda i: (0, i)),
              pl.BlockSpec((D, TW),
                           lambda i: (0, jnp.minimum(i + T_GRID, _HI_MAX)))],
    out_specs=pl.BlockSpec((TW, DP), lambda i: (i, 0)),
    out_shape=jax.ShapeDtypeStruct((H, DP), jnp.float32),
    compiler_params=pltpu.CompilerParams(
        dimension_semantics=("arbitrary",)),
)

_mesh = plsc.VectorSubcoreMesh(core_axis_name="c", subcore_axis_name="s")


@functools.partial(
    pl.kernel,
    mesh=_mesh,
    out_type=jax.ShapeDtypeStruct((SC_OUT,), jnp.float32),
    scratch_types=[
        pltpu.VMEM((2, IW), jnp.int32),
        pltpu.VMEM((2, CH, DP), jnp.float32),
        pltpu.VMEM((2, K * CH, DP), jnp.float32),
        pltpu.VMEM((2, CH, DP), jnp.float32),
        pltpu.VMEM((CH * 2 * L,), jnp.float32),
        pltpu.VMEM((K * CH * 2 * L,), jnp.float32),
        pltpu.SemaphoreType.DMA,
        pltpu.SemaphoreType.DMA,
        pltpu.SemaphoreType.DMA,
        pltpu.SemaphoreType.DMA,
    ],
)
def _sc_scores(table, idx_pk, hid, out,
               idx_v, lab_v, neg_v, hid_v, ps_v, ns_v,
               semi0, semi1, semg0, semg1):
    w = lax.axis_index("s") * NC + lax.axis_index("c")
    semi = [semi0, semi1]
    semg = [semg0, semg1]

    def issue_idx(c):
        b = c % 2
        return pltpu.async_copy(idx_pk.at[w * NCHUNK + c], idx_v.at[b],
                                semi[b])

    def issue_gathers(c):
        b = c % 2
        base = pl.multiple_of(w * R + c * CH, CH)
        cps = [pltpu.async_copy(table.at[idx_v.at[b, pl.ds(0, CH)]],
                                lab_v.at[b], semg[b])]
        for k in range(K):
            cps.append(pltpu.async_copy(
                table.at[idx_v.at[b, pl.ds((k + 1) * CH, CH)]],
                neg_v.at[b, pl.ds(k * CH, CH)], semg[b]))
        cps.append(pltpu.async_copy(hid.at[pl.ds(base, CH)], hid_v.at[b],
                                    semg[b]))
        return cps

    def compute(c):
        b = c % 2
        base = pl.multiple_of(w * R + c * CH, CH)

        def one_row(i):
            hs = [hid_v[b, i, pl.ds(t * L, L)] for t in range(NL)]

            def dots(e_ref, j):
                p = [hs[t] * e_ref[b, j, pl.ds(t * L, L)] for t in range(NL)]
                lo = (p[0] + p[1]) + (p[2] + p[3])
                hi = (p[4] + p[5]) + (p[6] + p[7])
                return lo, hi

            lo, hi = dots(lab_v, i)
            ps_v[pl.ds(i * 2 * L, L)] = lo
            ps_v[pl.ds(i * 2 * L + L, L)] = hi
            for k in range(K):
                j = i * K + k
                lo, hi = dots(neg_v, j)
                ns_v[pl.ds(j * 2 * L, L)] = lo
                ns_v[pl.ds(j * 2 * L + L, L)] = hi

        def row2(i2, _):
            one_row(i2 * 2)
            one_row(i2 * 2 + 1)
            return 0

        lax.fori_loop(0, CH // 2, row2, 0)
        pltpu.sync_copy(ps_v, out.at[pl.ds(base * 2 * L, CH * 2 * L)])
        pltpu.sync_copy(
            ns_v, out.at[pl.ds((N + base * K) * 2 * L, K * CH * 2 * L)])

    pend_idx = {0: issue_idx(0), 1: issue_idx(1)}
    pend_idx[0].wait()
    pend_g = {0: issue_gathers(0)}
    for c in range(NCHUNK):
        if c + 1 < NCHUNK:
            pend_idx[c + 1].wait()
            pend_g[c + 1] = issue_gathers(c + 1)
        for cp in pend_g[c]:
            cp.wait()
        if c + 2 < NCHUNK:
            pend_idx[c + 2] = issue_idx(c + 2)
        compute(c)


TC_ROWS = SC_OUT // 128
TC_BLK = 4096
SPR = 128 // (2 * L)


def _sel_mats():
    lane = lax.broadcasted_iota(jnp.int32, (128, SPR), 0)
    grp = lax.broadcasted_iota(jnp.int32, (128, SPR), 1)
    in_score = (lane // (2 * L)) == grp
    is_hi = (lane // L) % 2 == 1
    m_lo = (in_score & ~is_hi).astype(jnp.float32)
    m_hi = (in_score & is_hi).astype(jnp.float32)
    return m_lo, m_hi


def _tc_body(s_ref, par_ref, o_ref):
    b = pl.program_id(0)
    pos = b == 0
    sgn = jnp.where(pos, 1.0, -1.0)
    wgt = jnp.where(pos, 1.0, 1.0 / K)
    x = s_ref[...]
    m_lo, m_hi = _sel_mats()
    lo = lax.dot_general(x, m_lo, (((1,), (0,)), ((), ())),
                         preferred_element_type=jnp.float32)
    hi = lax.dot_general(x, m_hi, (((1,), (0,)), ((), ())),
                         preferred_element_type=jnp.float32)
    par = par_ref[...].astype(jnp.float32)
    score = lo + par * (hi - lo)
    y = sgn * score
    ls = jnp.minimum(y, 0.0) - jnp.log1p(jnp.exp(-jnp.abs(y)))

    @pl.when(b == 0)
    def _():
        o_ref[0, 0] = 0.0

    o_ref[0, 0] = o_ref[0, 0] - wgt * jnp.sum(ls)


_tc_loss = pl.pallas_call(
    _tc_body,
    grid=(TC_ROWS // TC_BLK,),
    in_specs=[pl.BlockSpec((TC_BLK, 128), lambda i: (i, 0)),
              pl.BlockSpec((TC_BLK, SPR), lambda i: (i, 0))],
    out_specs=pl.BlockSpec(memory_space=pltpu.SMEM, block_shape=(1, 1),
                           index_map=lambda i: (0, 0)),
    out_shape=jax.ShapeDtypeStruct((1, 1), jnp.float32),
    compiler_params=pltpu.CompilerParams(
        dimension_semantics=("arbitrary",)),
)


def kernel(hidden_state, label_idxes, neg_idxes, out_word_emb):
    lab = label_idxes.astype(jnp.int32)
    neg = neg_idxes.astype(jnp.int32)
    table2 = _repack(out_word_emb.T, out_word_emb.T)
    hid3 = jnp.concatenate([hidden_state, hidden_state], axis=1)
    lab_g = (lab & (H - 1)).reshape(NW, NCHUNK, CH)
    neg_g = (neg & (H - 1)).reshape(NW, NCHUNK, K * CH)
    idx_pk = jnp.concatenate([lab_g, neg_g], axis=2).reshape(NW * NCHUNK, IW)
    partials = _sc_scores(table2, idx_pk, hid3)
    par = jnp.concatenate([lab >> 19, neg >> 19]).reshape(TC_ROWS, SPR)
    loss = _tc_loss(partials.reshape(TC_ROWS, 128), par)
    return loss.reshape(())
